# baseline (speedup 1.0000x reference)
import jax
import jax.numpy as jnp
from jax.experimental import pallas as pl
from jax.experimental.pallas import tpu as pltpu

_R = 256
_K = 6


def _copy_kernel(x_ref, o_ref, buf, load_sem, store_sem):
    n = x_ref.shape[0] // _R

    def load(j):
        slot = j % _K
        pltpu.make_async_copy(
            x_ref.at[pl.ds(j * _R, _R), :],
            buf.at[pl.ds(slot * _R, _R), :],
            load_sem.at[slot],
        ).start()

    def store(j):
        slot = j % _K
        pltpu.make_async_copy(
            buf.at[pl.ds(slot * _R, _R), :],
            o_ref.at[pl.ds(j * _R, _R), :],
            store_sem.at[slot],
        ).start()

    def wait_load(j):
        slot = j % _K
        pltpu.make_async_copy(
            x_ref.at[pl.ds(j * _R, _R), :],
            buf.at[pl.ds(slot * _R, _R), :],
            load_sem.at[slot],
        ).wait()

    def wait_store(j):
        slot = j % _K
        pltpu.make_async_copy(
            buf.at[pl.ds(slot * _R, _R), :],
            o_ref.at[pl.ds(j * _R, _R), :],
            store_sem.at[slot],
        ).wait()

    for j in range(min(_K, n)):
        load(j)
    for j in range(n):
        wait_load(j)
        store(j)
        nj = j + _K
        if nj < n:
            wait_store(j)
            load(nj)
    for j in range(max(0, n - _K), n):
        wait_store(j)


def kernel(x):
    b, c, f = x.shape
    x2 = x.reshape(b * c, f)
    out = pl.pallas_call(
        _copy_kernel,
        out_shape=jax.ShapeDtypeStruct(x2.shape, x2.dtype),
        in_specs=[pl.BlockSpec(memory_space=pl.ANY)],
        out_specs=pl.BlockSpec(memory_space=pl.ANY),
        scratch_shapes=[
            pltpu.VMEM((_K * _R, f), x.dtype),
            pltpu.SemaphoreType.DMA((_K,)),
            pltpu.SemaphoreType.DMA((_K,)),
        ],
    )(x2)
    return out.reshape(b, c, f)
